# baseline (speedup 1.0000x reference)
import jax
import jax.numpy as jnp
from jax.experimental import pallas as pl
from jax.experimental.pallas import tpu as pltpu

N = 4096
D = 256
R = 256
NI = N // R


def _body(x1, x2, wgcn, wgat, asrc, adst, wproj, bproj, adj1, adj2,
          z1, z2, h1p, wh2, et1, et2, r, eemr):
    i = pl.program_id(0)

    @pl.when(i == 0)
    def _init():
        h1p[:] = jnp.dot(x1[:], wgcn[:],
                         preferred_element_type=jnp.float32).astype(jnp.bfloat16)
        wh = jnp.dot(x2[:], wgat[:], preferred_element_type=jnp.float32)
        wh2[:] = wh.astype(jnp.bfloat16)
        ss = jax.lax.dot_general(wh, asrc[:], (((1,), (1,)), ((), ())),
                                 preferred_element_type=jnp.float32)
        tt = jax.lax.dot_general(adst[:], wh, (((1,), (1,)), ((), ())),
                                 preferred_element_type=jnp.float32)
        e1 = jnp.exp(tt)
        e2 = jnp.exp(0.2 * tt)
        et1[:] = e1
        et2[:] = e2
        rr = jnp.exp(-0.8 * ss)
        r[:] = rr
        eemr[:] = jnp.maximum(jnp.max(e1), rr * jnp.max(e2))

    a1 = adj1[:]
    deg = jnp.sum(a1, axis=1, keepdims=True)
    n1 = jnp.dot(a1.astype(jnp.bfloat16), h1p[:],
                 preferred_element_type=jnp.float32)
    h1 = jnp.maximum(n1 / (deg + 1e-6), 0.0)
    z1[:] = jnp.dot(h1, wproj[:], preferred_element_type=jnp.float32) + bproj[:]

    rr = r[pl.ds(i * R, R), :]
    w = adj2[:] * jnp.maximum(et1[:], rr * et2[:])
    den = jnp.sum(w, axis=1, keepdims=True)
    n2 = jnp.dot(w.astype(jnp.bfloat16), wh2[:],
                 preferred_element_type=jnp.float32)
    h2 = n2 / (den + 1e-6 * eemr[pl.ds(i * R, R), :])
    h2 = jnp.where(h2 > 0, h2, jnp.exp(jnp.minimum(h2, 0.0)) - 1.0)
    z2[:] = jnp.dot(h2, wproj[:], preferred_element_type=jnp.float32) + bproj[:]


def _run(x1, x2, W_gcn, W_gat, a_src, a_dst, W_proj, b_proj, adj1, adj2,
         interpret=False):
    full = lambda i: (0, 0)
    return pl.pallas_call(
        _body,
        grid=(NI,),
        in_specs=[
            pl.BlockSpec((N, D), full),
            pl.BlockSpec((N, D), full),
            pl.BlockSpec((D, D), full),
            pl.BlockSpec((D, D), full),
            pl.BlockSpec((1, D), full),
            pl.BlockSpec((1, D), full),
            pl.BlockSpec((D, D), full),
            pl.BlockSpec((1, D), full),
            pl.BlockSpec((R, N), lambda i: (i, 0)),
            pl.BlockSpec((R, N), lambda i: (i, 0)),
        ],
        out_specs=[
            pl.BlockSpec((R, D), lambda i: (i, 0)),
            pl.BlockSpec((R, D), lambda i: (i, 0)),
        ],
        out_shape=[
            jax.ShapeDtypeStruct((N, D), jnp.float32),
            jax.ShapeDtypeStruct((N, D), jnp.float32),
        ],
        scratch_shapes=[
            pltpu.VMEM((N, D), jnp.bfloat16),
            pltpu.VMEM((N, D), jnp.bfloat16),
            pltpu.VMEM((1, N), jnp.float32),
            pltpu.VMEM((1, N), jnp.float32),
            pltpu.VMEM((N, 1), jnp.float32),
            pltpu.VMEM((N, 1), jnp.float32),
        ],
        interpret=interpret,
    )(x1, x2, W_gcn, W_gat, a_src, a_dst, W_proj, b_proj, adj1, adj2)


def kernel(x1, adj1, x2, adj2, W_gcn, W_gat, a_src, a_dst, W_proj, b_proj):
    z1, z2 = _run(x1, x2, W_gcn, W_gat,
                  a_src.reshape(1, D), a_dst.reshape(1, D),
                  W_proj, b_proj.reshape(1, D), adj1, adj2)
    return (z1, z2)


# speedup vs baseline: 1.1064x; 1.1064x over previous
import jax
import jax.numpy as jnp
from jax.experimental import pallas as pl
from jax.experimental.pallas import tpu as pltpu

N = 4096
D = 256
R = 1024
C = 1024
NI = N // R
NJ = N // C


def _body(x1, x2, wgcn, wgat, asrc, adst, wproj, bproj, adj1, adj2,
          z1, z2, h1p, wh2, et1, et2, r, eemr, acc1, acc2, deg, den):
    i = pl.program_id(0)
    j = pl.program_id(1)

    @pl.when((i == 0) & (j == 0))
    def _init():
        h1p[:] = jnp.dot(x1[:], wgcn[:],
                         preferred_element_type=jnp.float32).astype(jnp.bfloat16)
        wh = jnp.dot(x2[:], wgat[:], preferred_element_type=jnp.float32)
        wh2[:] = wh.astype(jnp.bfloat16)
        ss = jax.lax.dot_general(wh, asrc[:], (((1,), (1,)), ((), ())),
                                 preferred_element_type=jnp.float32)
        tt = jax.lax.dot_general(adst[:], wh, (((1,), (1,)), ((), ())),
                                 preferred_element_type=jnp.float32)
        e1 = jnp.exp(tt)
        e2 = jnp.exp(0.2 * tt)
        et1[:] = e1
        et2[:] = e2
        rr = jnp.exp(-0.8 * ss)
        r[:] = rr
        eemr[:] = jnp.maximum(jnp.max(e1), rr * jnp.max(e2))

    @pl.when(j == 0)
    def _reset():
        acc1[:] = jnp.zeros_like(acc1)
        acc2[:] = jnp.zeros_like(acc2)
        deg[:] = jnp.zeros_like(deg)
        den[:] = jnp.zeros_like(den)

    a1 = adj1[:]
    deg[:] = deg[:] + jnp.sum(a1, axis=1, keepdims=True)
    acc1[:] = acc1[:] + jnp.dot(a1.astype(jnp.bfloat16), h1p[pl.ds(j * C, C), :],
                                preferred_element_type=jnp.float32)

    rr = r[pl.ds(i * R, R), :]
    w = adj2[:] * jnp.maximum(et1[:, pl.ds(j * C, C)], rr * et2[:, pl.ds(j * C, C)])
    den[:] = den[:] + jnp.sum(w, axis=1, keepdims=True)
    acc2[:] = acc2[:] + jnp.dot(w.astype(jnp.bfloat16), wh2[pl.ds(j * C, C), :],
                                preferred_element_type=jnp.float32)

    @pl.when(j == NJ - 1)
    def _fin():
        h1 = jnp.maximum(acc1[:] / (deg[:] + 1e-6), 0.0)
        z1[:] = jnp.dot(h1, wproj[:], preferred_element_type=jnp.float32) + bproj[:]
        h2 = acc2[:] / (den[:] + 1e-6 * eemr[pl.ds(i * R, R), :])
        h2 = jnp.where(h2 > 0, h2, jnp.exp(jnp.minimum(h2, 0.0)) - 1.0)
        z2[:] = jnp.dot(h2, wproj[:], preferred_element_type=jnp.float32) + bproj[:]


def _run(x1, x2, W_gcn, W_gat, a_src, a_dst, W_proj, b_proj, adj1, adj2,
         interpret=False):
    full = lambda i, j: (0, 0)
    return pl.pallas_call(
        _body,
        grid=(NI, NJ),
        in_specs=[
            pl.BlockSpec((N, D), full),
            pl.BlockSpec((N, D), full),
            pl.BlockSpec((D, D), full),
            pl.BlockSpec((D, D), full),
            pl.BlockSpec((1, D), full),
            pl.BlockSpec((1, D), full),
            pl.BlockSpec((D, D), full),
            pl.BlockSpec((1, D), full),
            pl.BlockSpec((R, C), lambda i, j: (i, j)),
            pl.BlockSpec((R, C), lambda i, j: (i, j)),
        ],
        out_specs=[
            pl.BlockSpec((R, D), lambda i, j: (i, 0)),
            pl.BlockSpec((R, D), lambda i, j: (i, 0)),
        ],
        out_shape=[
            jax.ShapeDtypeStruct((N, D), jnp.float32),
            jax.ShapeDtypeStruct((N, D), jnp.float32),
        ],
        scratch_shapes=[
            pltpu.VMEM((N, D), jnp.bfloat16),
            pltpu.VMEM((N, D), jnp.bfloat16),
            pltpu.VMEM((1, N), jnp.float32),
            pltpu.VMEM((1, N), jnp.float32),
            pltpu.VMEM((N, 1), jnp.float32),
            pltpu.VMEM((N, 1), jnp.float32),
            pltpu.VMEM((R, D), jnp.float32),
            pltpu.VMEM((R, D), jnp.float32),
            pltpu.VMEM((R, 1), jnp.float32),
            pltpu.VMEM((R, 1), jnp.float32),
        ],
        interpret=interpret,
    )(x1, x2, W_gcn, W_gat, a_src, a_dst, W_proj, b_proj, adj1, adj2)


def kernel(x1, adj1, x2, adj2, W_gcn, W_gat, a_src, a_dst, W_proj, b_proj):
    z1, z2 = _run(x1, x2, W_gcn, W_gat,
                  a_src.reshape(1, D), a_dst.reshape(1, D),
                  W_proj, b_proj.reshape(1, D), adj1, adj2)
    return (z1, z2)
